# baseline (speedup 1.0000x reference)
import functools

import jax
import jax.numpy as jnp
from jax import lax
from jax.experimental import pallas as pl
from jax.experimental.pallas import tpu as pltpu
from jax.experimental.pallas import tpu_sc as plsc

NC, NS = 2, 16
NW = NC * NS
NQ = 32
D_GATE = 256
CHUNK = 128
PHASES = 5
TM = 4096


def _sc_gather_counts(gates_flat, qubits_flat, gate_table, tok0, mp):
    tpw = mp // NW
    nchunk = tpw // CHUNK
    mesh = plsc.VectorSubcoreMesh(core_axis_name="c", subcore_axis_name="s")

    @functools.partial(
        pl.kernel,
        mesh=mesh,
        out_type=(
            jax.ShapeDtypeStruct((mp, D_GATE), jnp.float32),
            jax.ShapeDtypeStruct((mp * NQ,), jnp.float32),
        ),
        scratch_types=[
            pltpu.VMEM((CHUNK,), jnp.int32),
            pltpu.VMEM((CHUNK, D_GATE), jnp.float32),
            pltpu.VMEM((CHUNK * NQ,), jnp.int32),
            pltpu.VMEM((CHUNK * NQ,), jnp.float32),
            pltpu.SemaphoreType.DMA,
        ],
        compiler_params=pltpu.CompilerParams(needs_layout_passes=False),
    )
    def k(gates_hbm, qubits_hbm, table_hbm, emb_hbm, counts_hbm,
          idx_v, rows_v, qub_v, cnt_v, sem):
        wid = lax.axis_index("s") * NC + lax.axis_index("c")
        lbase0 = wid * tpw

        ones = jnp.ones((16,), jnp.float32)
        zeros = jnp.zeros((16,), jnp.float32)

        def chunk_body(ci, carry):
            lbase = lbase0 + ci * CHUNK
            gbase = tok0 + lbase
            pltpu.sync_copy(gates_hbm.at[pl.ds(gbase, CHUNK)], idx_v)
            gather = pltpu.async_copy(table_hbm.at[idx_v], rows_v, sem)
            pltpu.sync_copy(qubits_hbm.at[pl.ds(gbase * NQ, CHUNK * NQ)],
                            qub_v)

            def zero_body(j, c):
                cnt_v[pl.ds(j * 16, 16)] = zeros
                return c
            lax.fori_loop(0, CHUNK * NQ // 16, zero_body, None, unroll=8)

            def tok_body(t, c):
                b = t * NQ
                q0 = qub_v[pl.ds(b, 16)]
                q1 = qub_v[pl.ds(b + 16, 16)]
                plsc.addupdate_scatter(cnt_v, [q0 + b], ones)
                plsc.addupdate_scatter(cnt_v, [q1 + b], ones)
                return c
            lax.fori_loop(0, CHUNK, tok_body, None, unroll=4)

            gather.wait()
            pltpu.sync_copy(rows_v, emb_hbm.at[pl.ds(lbase, CHUNK)])
            pltpu.sync_copy(cnt_v,
                            counts_hbm.at[pl.ds(lbase * NQ, CHUNK * NQ)])
            return carry

        lax.fori_loop(0, nchunk, chunk_body, None)

    return k(gates_flat, qubits_flat, gate_table)


def _tc_combine_phase(prev_out, gate_emb, counts, params, qubit_table, Wp,
                      bp2, Wf, bf2, m, mp, blk0, d_model):
    d4 = d_model // 4

    def body(*refs):
        if prev_out is None:
            (g_ref, c_ref, p_ref, qt_ref, wp_ref, wf_ref, bp_ref, bf_ref,
             o_ref) = refs
        else:
            (_prev, g_ref, c_ref, p_ref, qt_ref, wp_ref, wf_ref, bp_ref,
             bf_ref, o_ref) = refs
        wf = wf_ref[...]
        wq2 = jnp.dot(qt_ref[...], wf[D_GATE:D_GATE + d4],
                      preferred_element_type=jnp.float32)
        wp2 = jnp.dot(wp_ref[...], wf[D_GATE + d4:],
                      preferred_element_type=jnp.float32)
        bias = jnp.dot(bp_ref[...], wf[D_GATE + d4:],
                       preferred_element_type=jnp.float32) + bf_ref[...]
        acc = jnp.dot(g_ref[...], wf[:D_GATE],
                      preferred_element_type=jnp.float32)
        acc = acc + jnp.dot(c_ref[...], wq2,
                            preferred_element_type=jnp.float32) * (1.0 / NQ)
        acc = acc + jnp.dot(p_ref[...], wp2,
                            preferred_element_type=jnp.float32)
        o_ref[...] = acc + bias

    in_specs = [
        pl.BlockSpec((TM, D_GATE), lambda i: (i, 0)),
        pl.BlockSpec((TM, NQ), lambda i: (i, 0)),
        pl.BlockSpec((TM, 8), lambda i: (i + blk0, 0)),
        pl.BlockSpec((NQ, d_model // 4), lambda i: (0, 0)),
        pl.BlockSpec((8, d_model // 4), lambda i: (0, 0)),
        pl.BlockSpec((d_model, d_model), lambda i: (0, 0)),
        pl.BlockSpec((1, d_model // 4), lambda i: (0, 0)),
        pl.BlockSpec((1, d_model), lambda i: (0, 0)),
    ]
    args = [gate_emb, counts, params, qubit_table, Wp, Wf, bp2, bf2]
    aliases = {}
    if prev_out is not None:
        in_specs = [pl.BlockSpec((8, 128), lambda i: (0, 0))] + in_specs
        args = [prev_out] + args
        aliases = {0: 0}

    return pl.pallas_call(
        body,
        grid=(mp // TM,),
        in_specs=in_specs,
        out_specs=pl.BlockSpec((TM, d_model), lambda i: (i + blk0, 0)),
        out_shape=jax.ShapeDtypeStruct((m, d_model), jnp.float32),
        input_output_aliases=aliases,
    )(*args)


def kernel(gates, qubits, parameters, gate_table, qubit_table, Wp, bp, Wf,
           bf):
    b, s = gates.shape
    m = b * s
    d_model = Wf.shape[0]
    mp = m // PHASES

    gates_flat = gates.reshape(m).astype(jnp.int32)
    qubits_flat = qubits.reshape(m * NQ).astype(jnp.int32)
    params2 = parameters.reshape(m, parameters.shape[-1])
    bp2 = bp.reshape(1, -1)
    bf2 = bf.reshape(1, -1)

    embs, cnts = [], []
    for p in range(PHASES):
        e, c = _sc_gather_counts(gates_flat, qubits_flat, gate_table,
                                 p * mp, mp)
        embs.append(e)
        cnts.append(c.reshape(mp, NQ))

    out = None
    for p in range(PHASES):
        out = _tc_combine_phase(out, embs[p], cnts[p], params2,
                                qubit_table, Wp, bp2, Wf, bf2,
                                m, mp, p * (mp // TM), d_model)
    return out.reshape(b, s, d_model)


# speedup vs baseline: 1.0761x; 1.0761x over previous
import functools

import jax
import jax.numpy as jnp
from jax import lax
from jax.experimental import pallas as pl
from jax.experimental.pallas import tpu as pltpu
from jax.experimental.pallas import tpu_sc as plsc

NC, NS = 2, 16
NW = NC * NS
NQ = 32
D_GATE = 256
CHUNK = 128
TM = 4096


def _sc_gather_counts(gates_flat, qubits2, gate_table, m):
    tpw = m // NW
    nchunk = tpw // CHUNK
    mesh = plsc.VectorSubcoreMesh(core_axis_name="c", subcore_axis_name="s")

    @functools.partial(
        pl.kernel,
        mesh=mesh,
        out_type=(
            jax.ShapeDtypeStruct((m, D_GATE), jnp.float32),
            jax.ShapeDtypeStruct((m, NQ), jnp.float32),
        ),
        scratch_types=[
            pltpu.VMEM((CHUNK,), jnp.int32),
            pltpu.VMEM((CHUNK, D_GATE), jnp.float32),
            pltpu.VMEM((CHUNK, NQ), jnp.int32),
            pltpu.VMEM((CHUNK, NQ), jnp.float32),
            pltpu.SemaphoreType.DMA,
        ],
        compiler_params=pltpu.CompilerParams(needs_layout_passes=False),
    )
    def k(gates_hbm, qubits_hbm, table_hbm, emb_hbm, counts_hbm,
          idx_v, rows_v, qub_v, cnt_v, sem):
        wid = lax.axis_index("s") * NC + lax.axis_index("c")
        base0 = wid * tpw

        ones = jnp.ones((16,), jnp.float32)
        zeros = jnp.zeros((16,), jnp.float32)

        def chunk_body(ci, carry):
            base = base0 + ci * CHUNK
            pltpu.sync_copy(gates_hbm.at[pl.ds(base, CHUNK)], idx_v)
            gather = pltpu.async_copy(table_hbm.at[idx_v], rows_v, sem)
            pltpu.sync_copy(qubits_hbm.at[pl.ds(base, CHUNK)], qub_v)

            def tok_body(t, c):
                cnt_v[t, pl.ds(0, 16)] = zeros
                cnt_v[t, pl.ds(16, 16)] = zeros
                return c
            lax.fori_loop(0, CHUNK, tok_body, None, unroll=8)

            def tok_body2(t, c):
                q0 = qub_v[t, pl.ds(0, 16)]
                q1 = qub_v[t, pl.ds(16, 16)]
                tv = jnp.full((16,), t, jnp.int32)
                plsc.addupdate_scatter(cnt_v, [tv, q0], ones)
                plsc.addupdate_scatter(cnt_v, [tv, q1], ones)
                return c
            lax.fori_loop(0, CHUNK, tok_body2, None, unroll=4)

            gather.wait()
            pltpu.sync_copy(rows_v, emb_hbm.at[pl.ds(base, CHUNK)])
            pltpu.sync_copy(cnt_v, counts_hbm.at[pl.ds(base, CHUNK)])
            return carry

        lax.fori_loop(0, nchunk, chunk_body, None)

    return k(gates_flat, qubits2, gate_table)


def _tc_combine(gate_emb, counts, params, qubit_table, Wp, bp2, Wf, bf2,
                m, d_model, tm):
    d4 = d_model // 4

    def body(g_ref, c_ref, p_ref, qt_ref, wp_ref, wf_ref, bp_ref, bf_ref,
             o_ref):
        wf = wf_ref[...]
        wq2 = jnp.dot(qt_ref[...], wf[D_GATE:D_GATE + d4],
                      preferred_element_type=jnp.float32)
        wp2 = jnp.dot(wp_ref[...], wf[D_GATE + d4:],
                      preferred_element_type=jnp.float32)
        bias = jnp.dot(bp_ref[...], wf[D_GATE + d4:],
                       preferred_element_type=jnp.float32) + bf_ref[...]
        acc = jnp.dot(g_ref[...], wf[:D_GATE],
                      preferred_element_type=jnp.float32)
        acc = acc + jnp.dot(c_ref[...], wq2,
                            preferred_element_type=jnp.float32) * (1.0 / NQ)
        acc = acc + jnp.dot(p_ref[...], wp2,
                            preferred_element_type=jnp.float32)
        o_ref[...] = acc + bias

    return pl.pallas_call(
        body,
        grid=(m // tm,),
        in_specs=[
            pl.BlockSpec((tm, D_GATE), lambda i: (i, 0)),
            pl.BlockSpec((tm, NQ), lambda i: (i, 0)),
            pl.BlockSpec((tm, 8), lambda i: (i, 0)),
            pl.BlockSpec((NQ, d_model // 4), lambda i: (0, 0)),
            pl.BlockSpec((8, d_model // 4), lambda i: (0, 0)),
            pl.BlockSpec((d_model, d_model), lambda i: (0, 0)),
            pl.BlockSpec((1, d_model // 4), lambda i: (0, 0)),
            pl.BlockSpec((1, d_model), lambda i: (0, 0)),
        ],
        out_specs=pl.BlockSpec((tm, d_model), lambda i: (i, 0)),
        out_shape=jax.ShapeDtypeStruct((m, d_model), jnp.float32),
    )(gate_emb, counts, params, qubit_table, Wp, Wf, bp2, bf2)


def kernel(gates, qubits, parameters, gate_table, qubit_table, Wp, bp, Wf,
           bf):
    b, s = gates.shape
    m = b * s
    d_model = Wf.shape[0]

    gates_flat = gates.reshape(m).astype(jnp.int32)
    qubits2 = qubits.reshape(m, NQ).astype(jnp.int32)
    params2 = parameters.reshape(m, parameters.shape[-1])

    gate_emb, counts2 = _sc_gather_counts(gates_flat, qubits2, gate_table,
                                          m)

    out = _tc_combine(gate_emb, counts2, params2, qubit_table, Wp,
                      bp.reshape(1, -1), Wf, bf.reshape(1, -1),
                      m, d_model, tm=TM)
    return out.reshape(b, s, d_model)


# speedup vs baseline: 1.2130x; 1.1273x over previous
import functools

import jax
import jax.numpy as jnp
from jax import lax
from jax.experimental import pallas as pl
from jax.experimental.pallas import tpu as pltpu
from jax.experimental.pallas import tpu_sc as plsc

NC, NS = 2, 16
NW = NC * NS
NQ = 32
D_GATE = 256
CHUNK = 128
PHASES = 5
TM = 4096


def _sc_gather_counts(gates_flat, qubits2, gate_table, tok0, mp):
    tpw = mp // NW
    nchunk = tpw // CHUNK
    mesh = plsc.VectorSubcoreMesh(core_axis_name="c", subcore_axis_name="s")

    @functools.partial(
        pl.kernel,
        mesh=mesh,
        out_type=(
            jax.ShapeDtypeStruct((mp, D_GATE), jnp.float32),
            jax.ShapeDtypeStruct((mp, NQ), jnp.float32),
        ),
        scratch_types=[
            pltpu.VMEM((CHUNK,), jnp.int32),
            pltpu.VMEM((CHUNK, D_GATE), jnp.float32),
            pltpu.VMEM((CHUNK, NQ), jnp.int32),
            pltpu.VMEM((CHUNK, NQ), jnp.float32),
            pltpu.SemaphoreType.DMA,
        ],
        compiler_params=pltpu.CompilerParams(needs_layout_passes=False),
    )
    def k(gates_hbm, qubits_hbm, table_hbm, emb_hbm, counts_hbm,
          idx_v, rows_v, qub_v, cnt_v, sem):
        wid = lax.axis_index("s") * NC + lax.axis_index("c")
        lbase0 = wid * tpw

        ones = jnp.ones((16,), jnp.float32)
        zeros = jnp.zeros((16,), jnp.float32)

        def chunk_body(ci, carry):
            lbase = lbase0 + ci * CHUNK
            gbase = tok0 + lbase
            pltpu.sync_copy(gates_hbm.at[pl.ds(gbase, CHUNK)], idx_v)
            gather = pltpu.async_copy(table_hbm.at[idx_v], rows_v, sem)
            pltpu.sync_copy(qubits_hbm.at[pl.ds(gbase, CHUNK)], qub_v)

            def tok_body(t, c):
                cnt_v[t, pl.ds(0, 16)] = zeros
                cnt_v[t, pl.ds(16, 16)] = zeros
                return c
            lax.fori_loop(0, CHUNK, tok_body, None, unroll=8)

            def tok_body2(t, c):
                q0 = qub_v[t, pl.ds(0, 16)]
                q1 = qub_v[t, pl.ds(16, 16)]
                tv = jnp.full((16,), t, jnp.int32)
                plsc.addupdate_scatter(cnt_v, [tv, q0], ones)
                plsc.addupdate_scatter(cnt_v, [tv, q1], ones)
                return c
            lax.fori_loop(0, CHUNK, tok_body2, None, unroll=4)

            gather.wait()
            pltpu.sync_copy(rows_v, emb_hbm.at[pl.ds(lbase, CHUNK)])
            pltpu.sync_copy(cnt_v, counts_hbm.at[pl.ds(lbase, CHUNK)])
            return carry

        lax.fori_loop(0, nchunk, chunk_body, None)

    return k(gates_flat, qubits2, gate_table)


def _tc_combine_phase(prev_out, gate_emb, counts, params, qubit_table, Wp,
                      bp2, Wf, bf2, m, mp, blk0, d_model):
    d4 = d_model // 4

    def body(*refs):
        if prev_out is None:
            (g_ref, c_ref, p_ref, qt_ref, wp_ref, wf_ref, bp_ref, bf_ref,
             o_ref) = refs
        else:
            (_prev, g_ref, c_ref, p_ref, qt_ref, wp_ref, wf_ref, bp_ref,
             bf_ref, o_ref) = refs
        wf = wf_ref[...]
        wq2 = jnp.dot(qt_ref[...], wf[D_GATE:D_GATE + d4],
                      preferred_element_type=jnp.float32)
        wp2 = jnp.dot(wp_ref[...], wf[D_GATE + d4:],
                      preferred_element_type=jnp.float32)
        bias = jnp.dot(bp_ref[...], wf[D_GATE + d4:],
                       preferred_element_type=jnp.float32) + bf_ref[...]
        acc = jnp.dot(g_ref[...], wf[:D_GATE],
                      preferred_element_type=jnp.float32)
        acc = acc + jnp.dot(c_ref[...], wq2,
                            preferred_element_type=jnp.float32) * (1.0 / NQ)
        acc = acc + jnp.dot(p_ref[...], wp2,
                            preferred_element_type=jnp.float32)
        o_ref[...] = acc + bias

    in_specs = [
        pl.BlockSpec((TM, D_GATE), lambda i: (i, 0)),
        pl.BlockSpec((TM, NQ), lambda i: (i, 0)),
        pl.BlockSpec((TM, 8), lambda i: (i + blk0, 0)),
        pl.BlockSpec((NQ, d_model // 4), lambda i: (0, 0)),
        pl.BlockSpec((8, d_model // 4), lambda i: (0, 0)),
        pl.BlockSpec((d_model, d_model), lambda i: (0, 0)),
        pl.BlockSpec((1, d_model // 4), lambda i: (0, 0)),
        pl.BlockSpec((1, d_model), lambda i: (0, 0)),
    ]
    args = [gate_emb, counts, params, qubit_table, Wp, Wf, bp2, bf2]
    aliases = {}
    if prev_out is not None:
        in_specs = [pl.BlockSpec((8, 128), lambda i: (0, 0))] + in_specs
        args = [prev_out] + args
        aliases = {0: 0}

    return pl.pallas_call(
        body,
        grid=(mp // TM,),
        in_specs=in_specs,
        out_specs=pl.BlockSpec((TM, d_model), lambda i: (i + blk0, 0)),
        out_shape=jax.ShapeDtypeStruct((m, d_model), jnp.float32),
        input_output_aliases=aliases,
    )(*args)


def kernel(gates, qubits, parameters, gate_table, qubit_table, Wp, bp, Wf,
           bf):
    b, s = gates.shape
    m = b * s
    d_model = Wf.shape[0]
    mp = m // PHASES

    gates_flat = gates.reshape(m).astype(jnp.int32)
    qubits2 = qubits.reshape(m, NQ).astype(jnp.int32)
    params2 = parameters.reshape(m, parameters.shape[-1])
    bp2 = bp.reshape(1, -1)
    bf2 = bf.reshape(1, -1)

    out = None
    pending = []
    for p in range(PHASES):
        g_in = gates_flat
        if p >= 2:
            g_in, _ = lax.optimization_barrier((gates_flat, pending[p - 2]))
        e, c = _sc_gather_counts(g_in, qubits2, gate_table, p * mp, mp)
        out = _tc_combine_phase(out, e, c, params2, qubit_table, Wp, bp2,
                                Wf, bf2, m, mp, p * (mp // TM), d_model)
        pending.append(out)
    return out.reshape(b, s, d_model)


# speedup vs baseline: 1.2598x; 1.0385x over previous
import functools

import jax
import jax.numpy as jnp
from jax import lax
from jax.experimental import pallas as pl
from jax.experimental.pallas import tpu as pltpu
from jax.experimental.pallas import tpu_sc as plsc

NC, NS = 2, 16
NW = NC * NS
NQ = 32
D_GATE = 256
CHUNK = 128
PHASES = 5
TM = 4096


def _sc_gather_counts(gates_flat, qubits2, gate_table, tok0, mp):
    tpw = mp // NW
    nchunk = tpw // CHUNK
    mesh = plsc.VectorSubcoreMesh(core_axis_name="c", subcore_axis_name="s")

    @functools.partial(
        pl.kernel,
        mesh=mesh,
        out_type=(
            jax.ShapeDtypeStruct((mp, D_GATE // 2), jnp.int32),
            jax.ShapeDtypeStruct((mp, NQ), jnp.float32),
        ),
        scratch_types=[
            pltpu.VMEM((CHUNK,), jnp.int32),
            pltpu.VMEM((CHUNK, D_GATE // 2), jnp.int32),
            pltpu.VMEM((CHUNK, NQ), jnp.int32),
            pltpu.VMEM((CHUNK, NQ), jnp.float32),
            pltpu.SemaphoreType.DMA,
        ],
        compiler_params=pltpu.CompilerParams(needs_layout_passes=False),
    )
    def k(gates_hbm, qubits_hbm, table_hbm, emb_hbm, counts_hbm,
          idx_v, rows_v, qub_v, cnt_v, sem):
        wid = lax.axis_index("s") * NC + lax.axis_index("c")
        lbase0 = wid * tpw

        ones = jnp.ones((16,), jnp.float32)
        zeros = jnp.zeros((16,), jnp.float32)

        def chunk_body(ci, carry):
            lbase = lbase0 + ci * CHUNK
            gbase = tok0 + lbase
            pltpu.sync_copy(gates_hbm.at[pl.ds(gbase, CHUNK)], idx_v)
            gather = pltpu.async_copy(table_hbm.at[idx_v], rows_v, sem)
            pltpu.sync_copy(qubits_hbm.at[pl.ds(gbase, CHUNK)], qub_v)

            def tok_body(t, c):
                cnt_v[t, pl.ds(0, 16)] = zeros
                cnt_v[t, pl.ds(16, 16)] = zeros
                return c
            lax.fori_loop(0, CHUNK, tok_body, None, unroll=8)

            def tok_body2(t, c):
                q0 = qub_v[t, pl.ds(0, 16)]
                q1 = qub_v[t, pl.ds(16, 16)]
                tv = jnp.full((16,), t, jnp.int32)
                plsc.addupdate_scatter(cnt_v, [tv, q0], ones)
                plsc.addupdate_scatter(cnt_v, [tv, q1], ones)
                return c
            lax.fori_loop(0, CHUNK, tok_body2, None, unroll=4)

            gather.wait()
            pltpu.sync_copy(rows_v, emb_hbm.at[pl.ds(lbase, CHUNK)])
            pltpu.sync_copy(cnt_v, counts_hbm.at[pl.ds(lbase, CHUNK)])
            return carry

        lax.fori_loop(0, nchunk, chunk_body, None)

    return k(gates_flat, qubits2, gate_table)


def _tc_combine_phase(prev_out, gate_emb, counts, params, qubit_table, Wp,
                      bp2, Wf, bf2, m, mp, blk0, d_model):
    d4 = d_model // 4

    def body(*refs):
        if prev_out is None:
            (g_ref, c_ref, p_ref, qt_ref, wp_ref, wf_ref, bp_ref, bf_ref,
             o_ref) = refs
        else:
            (_prev, g_ref, c_ref, p_ref, qt_ref, wp_ref, wf_ref, bp_ref,
             bf_ref, o_ref) = refs
        wf = wf_ref[...]
        wq2 = jnp.dot(qt_ref[...], wf[D_GATE:D_GATE + d4],
                      preferred_element_type=jnp.float32)
        wp2 = jnp.dot(wp_ref[...], wf[D_GATE + d4:],
                      preferred_element_type=jnp.float32)
        bias = jnp.dot(bp_ref[...], wf[D_GATE + d4:],
                       preferred_element_type=jnp.float32) + bf_ref[...]
        u = lax.bitcast_convert_type(g_ref[...], jnp.uint32)
        g_lo = lax.bitcast_convert_type((u & 0xFFFF).astype(jnp.uint16),
                                        jnp.bfloat16)
        g_hi = lax.bitcast_convert_type((u >> 16).astype(jnp.uint16),
                                        jnp.bfloat16)
        dg2 = D_GATE // 2
        acc = jnp.dot(g_lo, wf[:dg2].astype(jnp.bfloat16),
                      preferred_element_type=jnp.float32)
        acc = acc + jnp.dot(g_hi, wf[dg2:D_GATE].astype(jnp.bfloat16),
                            preferred_element_type=jnp.float32)
        acc = acc + jnp.dot(c_ref[...], wq2,
                            preferred_element_type=jnp.float32) * (1.0 / NQ)
        acc = acc + jnp.dot(p_ref[...], wp2,
                            preferred_element_type=jnp.float32)
        o_ref[...] = acc + bias

    in_specs = [
        pl.BlockSpec((TM, D_GATE // 2), lambda i: (i, 0)),
        pl.BlockSpec((TM, NQ), lambda i: (i, 0)),
        pl.BlockSpec((TM, 8), lambda i: (i + blk0, 0)),
        pl.BlockSpec((NQ, d_model // 4), lambda i: (0, 0)),
        pl.BlockSpec((8, d_model // 4), lambda i: (0, 0)),
        pl.BlockSpec((d_model, d_model), lambda i: (0, 0)),
        pl.BlockSpec((1, d_model // 4), lambda i: (0, 0)),
        pl.BlockSpec((1, d_model), lambda i: (0, 0)),
    ]
    args = [gate_emb, counts, params, qubit_table, Wp, Wf, bp2, bf2]
    aliases = {}
    if prev_out is not None:
        in_specs = [pl.BlockSpec((8, 128), lambda i: (0, 0))] + in_specs
        args = [prev_out] + args
        aliases = {0: 0}

    return pl.pallas_call(
        body,
        grid=(mp // TM,),
        in_specs=in_specs,
        out_specs=pl.BlockSpec((TM, d_model), lambda i: (i + blk0, 0)),
        out_shape=jax.ShapeDtypeStruct((m, d_model), jnp.float32),
        input_output_aliases=aliases,
    )(*args)


def kernel(gates, qubits, parameters, gate_table, qubit_table, Wp, bp, Wf,
           bf):
    b, s = gates.shape
    m = b * s
    d_model = Wf.shape[0]
    mp = m // PHASES

    gates_flat = gates.reshape(m).astype(jnp.int32)
    qubits2 = qubits.reshape(m, NQ).astype(jnp.int32)
    tb = gate_table.astype(jnp.bfloat16)
    d2 = D_GATE // 2
    lo = lax.bitcast_convert_type(tb[:, :d2], jnp.uint16).astype(jnp.uint32)
    hi = lax.bitcast_convert_type(tb[:, d2:], jnp.uint16).astype(jnp.uint32)
    gate_table_pk = lax.bitcast_convert_type(lo | (hi << 16), jnp.int32)
    params2 = parameters.reshape(m, parameters.shape[-1])
    bp2 = bp.reshape(1, -1)
    bf2 = bf.reshape(1, -1)

    out = None
    pending = []
    for p in range(PHASES):
        g_in = gates_flat
        if p >= 2:
            g_in, _ = lax.optimization_barrier((gates_flat, pending[p - 2]))
        e, c = _sc_gather_counts(g_in, qubits2, gate_table_pk, p * mp, mp)
        out = _tc_combine_phase(out, e, c, params2, qubit_table, Wp, bp2,
                                Wf, bf2, m, mp, p * (mp // TM), d_model)
        pending.append(out)
    return out.reshape(b, s, d_model)


# speedup vs baseline: 1.3352x; 1.0599x over previous
import functools

import jax
import jax.numpy as jnp
from jax import lax
from jax.experimental import pallas as pl
from jax.experimental.pallas import tpu as pltpu
from jax.experimental.pallas import tpu_sc as plsc

NC, NS = 2, 16
NW = NC * NS
NQ = 32
D_GATE = 256
CHUNK = 128
PHASES = 5
TM = 4096


def _sc_gather_counts(gates_flat, qubits2, gate_table, tok0, mp):
    tpw = mp // NW
    nchunk = tpw // CHUNK
    mesh = plsc.VectorSubcoreMesh(core_axis_name="c", subcore_axis_name="s")

    @functools.partial(
        pl.kernel,
        mesh=mesh,
        out_type=(
            jax.ShapeDtypeStruct((mp, D_GATE // 2), jnp.int32),
            jax.ShapeDtypeStruct((mp, NQ), jnp.float32),
        ),
        scratch_types=[
            pltpu.VMEM((CHUNK,), jnp.int32),
            pltpu.VMEM((CHUNK, D_GATE // 2), jnp.int32),
            pltpu.VMEM((CHUNK, NQ), jnp.int32),
            pltpu.VMEM((CHUNK, NQ), jnp.float32),
            pltpu.SemaphoreType.DMA,
        ],
        compiler_params=pltpu.CompilerParams(needs_layout_passes=False),
    )
    def k(gates_hbm, qubits_hbm, table_hbm, emb_hbm, counts_hbm,
          idx_v, rows_v, qub_v, cnt_v, sem):
        wid = lax.axis_index("s") * NC + lax.axis_index("c")
        lbase0 = wid * tpw

        ones = jnp.ones((16,), jnp.float32)
        zeros = jnp.zeros((16,), jnp.float32)

        def chunk_body(ci, carry):
            lbase = lbase0 + ci * CHUNK
            gbase = tok0 + lbase
            pltpu.sync_copy(gates_hbm.at[pl.ds(gbase, CHUNK)], idx_v)
            gather = pltpu.async_copy(table_hbm.at[idx_v], rows_v, sem)
            pltpu.sync_copy(qubits_hbm.at[pl.ds(gbase, CHUNK)], qub_v)

            def tok_body(t, c):
                cnt_v[t, pl.ds(0, 16)] = zeros
                cnt_v[t, pl.ds(16, 16)] = zeros
                return c
            lax.fori_loop(0, CHUNK, tok_body, None, unroll=8)

            def tok_body2(t, c):
                q0 = qub_v[t, pl.ds(0, 16)]
                q1 = qub_v[t, pl.ds(16, 16)]
                tv = jnp.full((16,), t, jnp.int32)
                plsc.addupdate_scatter(cnt_v, [tv, q0], ones)
                plsc.addupdate_scatter(cnt_v, [tv, q1], ones)
                return c
            lax.fori_loop(0, CHUNK, tok_body2, None, unroll=4)

            gather.wait()
            pltpu.sync_copy(rows_v, emb_hbm.at[pl.ds(lbase, CHUNK)])
            pltpu.sync_copy(cnt_v, counts_hbm.at[pl.ds(lbase, CHUNK)])
            return carry

        lax.fori_loop(0, nchunk, chunk_body, None)

    return k(gates_flat, qubits2, gate_table)


def _tc_combine_phase(prev_out, gate_emb, counts, params, qubit_table, Wp,
                      bp2, Wf, bf2, m, mp, blk0, d_model):
    d4 = d_model // 4

    def body(*refs):
        if prev_out is None:
            (g_ref, c_ref, p_ref, qt_ref, wp_ref, wf_ref, bp_ref, bf_ref,
             o_ref) = refs
        else:
            (_prev, g_ref, c_ref, p_ref, qt_ref, wp_ref, wf_ref, bp_ref,
             bf_ref, o_ref) = refs
        wf = wf_ref[...]
        wq2 = jnp.dot(qt_ref[...], wf[D_GATE:D_GATE + d4],
                      preferred_element_type=jnp.float32)
        wp2 = jnp.dot(wp_ref[...], wf[D_GATE + d4:],
                      preferred_element_type=jnp.float32)
        bias = jnp.dot(bp_ref[...], wf[D_GATE + d4:],
                       preferred_element_type=jnp.float32) + bf_ref[...]
        u = lax.bitcast_convert_type(g_ref[...], jnp.uint32)
        g_lo = lax.bitcast_convert_type((u & 0xFFFF).astype(jnp.uint16),
                                        jnp.bfloat16)
        g_hi = lax.bitcast_convert_type((u >> 16).astype(jnp.uint16),
                                        jnp.bfloat16)
        dg2 = D_GATE // 2
        acc = jnp.dot(g_lo, wf[:dg2].astype(jnp.bfloat16),
                      preferred_element_type=jnp.float32)
        acc = acc + jnp.dot(g_hi, wf[dg2:D_GATE].astype(jnp.bfloat16),
                            preferred_element_type=jnp.float32)
        acc = acc + jnp.dot(c_ref[...], wq2,
                            preferred_element_type=jnp.float32) * (1.0 / NQ)
        acc = acc + jnp.dot(p_ref[...], wp2,
                            preferred_element_type=jnp.float32)
        o_ref[...] = acc + bias

    in_specs = [
        pl.BlockSpec((TM, D_GATE // 2), lambda i: (i, 0)),
        pl.BlockSpec((TM, NQ), lambda i: (i, 0)),
        pl.BlockSpec((TM, 8), lambda i: (i + blk0, 0)),
        pl.BlockSpec((NQ, d_model // 4), lambda i: (0, 0)),
        pl.BlockSpec((8, d_model // 4), lambda i: (0, 0)),
        pl.BlockSpec((d_model, d_model), lambda i: (0, 0)),
        pl.BlockSpec((1, d_model // 4), lambda i: (0, 0)),
        pl.BlockSpec((1, d_model), lambda i: (0, 0)),
    ]
    args = [gate_emb, counts, params, qubit_table, Wp, Wf, bp2, bf2]
    aliases = {}
    if prev_out is not None:
        in_specs = [pl.BlockSpec((8, 128), lambda i: (0, 0))] + in_specs
        args = [prev_out] + args
        aliases = {0: 0}

    return pl.pallas_call(
        body,
        grid=(mp // TM,),
        in_specs=in_specs,
        out_specs=pl.BlockSpec((TM, d_model), lambda i: (i + blk0, 0)),
        out_shape=jax.ShapeDtypeStruct((m, d_model), jnp.float32),
        input_output_aliases=aliases,
    )(*args)


def kernel(gates, qubits, parameters, gate_table, qubit_table, Wp, bp, Wf,
           bf):
    b, s = gates.shape
    m = b * s
    d_model = Wf.shape[0]
    mp = m // PHASES

    gates_flat = gates.reshape(m).astype(jnp.int32)
    qubits2 = qubits.reshape(m, NQ).astype(jnp.int32)
    d2 = D_GATE // 2

    def _rne_bf16_bits(x):
        u = lax.bitcast_convert_type(x, jnp.uint32)
        return (u + 0x7FFF + ((u >> 16) & 1)) >> 16

    lo = _rne_bf16_bits(gate_table[:, :d2])
    hi = _rne_bf16_bits(gate_table[:, d2:])
    gate_table_pk = lax.bitcast_convert_type(lo | (hi << 16), jnp.int32)
    params2 = parameters.reshape(m, parameters.shape[-1])
    bp2 = bp.reshape(1, -1)
    bf2 = bf.reshape(1, -1)

    out = None
    pending = []
    for p in range(PHASES):
        g_in = gates_flat
        if p >= 2:
            g_in, _ = lax.optimization_barrier((gates_flat, pending[p - 2]))
        e, c = _sc_gather_counts(g_in, qubits2, gate_table_pk, p * mp, mp)
        out = _tc_combine_phase(out, e, c, params2, qubit_table, Wp, bp2,
                                Wf, bf2, m, mp, p * (mp // TM), d_model)
        pending.append(out)
    return out.reshape(b, s, d_model)
